# baseline (speedup 1.0000x reference)
import functools

import jax
import jax.numpy as jnp
from jax import lax
from jax.experimental import pallas as pl
from jax.experimental.pallas import tpu as pltpu
from jax.experimental.pallas import tpu_sc as plsc

_LANES = 16


def _split_blocks(chunk):
  assert chunk % _LANES == 0
  vecs = chunk // _LANES
  for nb in range(max(1, vecs // 1024), vecs + 1):
    if vecs % nb == 0 and vecs // nb <= 1024:
      return vecs // nb * _LANES, nb
  return _LANES, vecs


def _wid():
  return lax.axis_index("s") * 2 + lax.axis_index("c")


def _minmax_body(nblocks, blk, chunk, tgt, out, buf0, buf1, stage, sem0, sem1):
  wid = _wid()
  base = wid * chunk
  bufs = (buf0, buf1)
  sems = (sem0, sem1)
  vpb = blk // _LANES

  copies = [None, None]
  copies[0] = pltpu.async_copy(tgt.at[pl.ds(base, blk)], buf0, sem0)

  vmin = jnp.full((_LANES,), jnp.inf, jnp.float32)
  vmax = jnp.full((_LANES,), -jnp.inf, jnp.float32)

  for b in range(nblocks):
    cur = b % 2
    if b + 1 < nblocks:
      nxt = (b + 1) % 2
      copies[nxt] = pltpu.async_copy(
          tgt.at[pl.ds(base + (b + 1) * blk, blk)], bufs[nxt], sems[nxt])
    copies[cur].wait()
    buf = bufs[cur]

    def body(i, carry):
      mn, mx = carry
      t = buf[pl.ds(i * _LANES, _LANES)]
      return jnp.minimum(mn, t), jnp.maximum(mx, t)

    vmin, vmax = lax.fori_loop(0, vpb, body, (vmin, vmax))

  stage[...] = vmin
  pltpu.sync_copy(stage, out.at[0, wid])
  stage[...] = vmax
  pltpu.sync_copy(stage, out.at[1, wid])


def _wmse_body(nblocks, blk, chunk, kbins,
               pred, tgt, wflat, minv, scalev, gbase, out,
               tb0, tb1, pb0, pb1, wv, mv, sv, gv, stage,
               tsem0, tsem1, psem0, psem1):
  wid = _wid()
  base = wid * chunk
  tbufs = (tb0, tb1)
  pbufs = (pb0, pb1)
  tsems = (tsem0, tsem1)
  psems = (psem0, psem1)
  vpb = blk // _LANES

  pltpu.sync_copy(wflat, wv)
  pltpu.sync_copy(minv, mv)
  pltpu.sync_copy(scalev, sv)
  pltpu.sync_copy(gbase, gv)
  m = mv[...]
  s = sv[...]
  g = gv[...]

  tcopies = [None, None]
  pcopies = [None, None]
  tcopies[0] = pltpu.async_copy(tgt.at[pl.ds(base, blk)], tb0, tsem0)
  pcopies[0] = pltpu.async_copy(pred.at[pl.ds(base, blk)], pb0, psem0)

  acc = jnp.zeros((_LANES,), jnp.float32)

  for b in range(nblocks):
    cur = b % 2
    if b + 1 < nblocks:
      nxt = (b + 1) % 2
      off = base + (b + 1) * blk
      tcopies[nxt] = pltpu.async_copy(
          tgt.at[pl.ds(off, blk)], tbufs[nxt], tsems[nxt])
      pcopies[nxt] = pltpu.async_copy(
          pred.at[pl.ds(off, blk)], pbufs[nxt], psems[nxt])
    tcopies[cur].wait()
    pcopies[cur].wait()
    tbuf = tbufs[cur]
    pbuf = pbufs[cur]

    def body(i, acc):
      t = tbuf[pl.ds(i * _LANES, _LANES)]
      p = pbuf[pl.ds(i * _LANES, _LANES)]
      d = p - t
      sq = d * d
      u = (t - m) * s
      bi = u.astype(jnp.int32)
      bi = jnp.minimum(jnp.maximum(bi, 0), kbins - 1)
      w = plsc.load_gather(wv, [bi + g])
      return acc + sq * w

    acc = lax.fori_loop(0, vpb, body, acc)

  stage[...] = acc
  pltpu.sync_copy(stage, out.at[wid])


@jax.jit
def kernel(pred, target, weights):
  n, n_genes = target.shape
  kbins = weights.shape[1]
  total = n * n_genes

  info = plsc.get_sparse_core_info()
  nw = info.num_cores * info.num_subcores
  assert total % (nw * _LANES) == 0
  chunk = total // nw
  blk, nblocks = _split_blocks(chunk)

  mesh = plsc.VectorSubcoreMesh(core_axis_name="c", subcore_axis_name="s")
  tflat = target.reshape(-1)
  pflat = pred.reshape(-1)

  minmax = pl.kernel(
      functools.partial(_minmax_body, nblocks, blk, chunk),
      out_type=jax.ShapeDtypeStruct((2, nw, _LANES), jnp.float32),
      mesh=mesh,
      scratch_types=[
          pltpu.VMEM((blk,), jnp.float32),
          pltpu.VMEM((blk,), jnp.float32),
          pltpu.VMEM((_LANES,), jnp.float32),
          pltpu.SemaphoreType.DMA,
          pltpu.SemaphoreType.DMA,
      ],
  )(tflat)

  lane_min = minmax[0].min(axis=0)
  lane_max = minmax[1].max(axis=0)
  gmin = lane_min.reshape(-1, n_genes).min(axis=0)
  gmax = lane_max.reshape(-1, n_genes).max(axis=0)
  scale = kbins / (gmax - gmin)
  minv = jnp.tile(gmin, _LANES // n_genes)
  scalev = jnp.tile(scale, _LANES // n_genes)
  gbase = (jnp.arange(_LANES, dtype=jnp.int32) % n_genes) * kbins

  partial = pl.kernel(
      functools.partial(_wmse_body, nblocks, blk, chunk, kbins),
      out_type=jax.ShapeDtypeStruct((nw, _LANES), jnp.float32),
      mesh=mesh,
      scratch_types=[
          pltpu.VMEM((blk,), jnp.float32),
          pltpu.VMEM((blk,), jnp.float32),
          pltpu.VMEM((blk,), jnp.float32),
          pltpu.VMEM((blk,), jnp.float32),
          pltpu.VMEM((n_genes * kbins,), jnp.float32),
          pltpu.VMEM((_LANES,), jnp.float32),
          pltpu.VMEM((_LANES,), jnp.float32),
          pltpu.VMEM((_LANES,), jnp.int32),
          pltpu.VMEM((_LANES,), jnp.float32),
          pltpu.SemaphoreType.DMA,
          pltpu.SemaphoreType.DMA,
          pltpu.SemaphoreType.DMA,
          pltpu.SemaphoreType.DMA,
      ],
      compiler_params=pltpu.CompilerParams(needs_layout_passes=False),
  )(pflat, tflat, weights.reshape(-1), minv, scalev, gbase)

  return jnp.sum(partial) / (n_genes * n)


# speedup vs baseline: 50.1181x; 50.1181x over previous
import functools

import jax
import jax.numpy as jnp
from jax import lax
from jax.experimental import pallas as pl
from jax.experimental.pallas import tpu as pltpu
from jax.experimental.pallas import tpu_sc as plsc

_L = 16
_TW = 128
_NW = 32


def _wid():
  return lax.axis_index("s") * 2 + lax.axis_index("c")


def _tile_split(num_tiles):
  tps, rem = divmod(num_tiles, _NW)
  t = 1
  for d in range(1, tps + 1):
    if tps % d == 0 and d <= 64:
      t = d
  return tps, rem, t, tps // t


def _minmax_body(n_genes, tps, rem, tpb, nblk, num_tiles,
                 tgt, out, b0, b1, tail, stmin, stmax, sem0, sem1, semt):
  wid = _wid()
  base = wid * tps + jnp.minimum(wid, rem)
  bufs = (b0, b1)
  sems = (sem0, sem1)

  tailcp = pltpu.async_copy(
      tgt.at[pl.ds(jnp.minimum(base + tps, num_tiles - 1), 1)], tail, semt)
  cps = [None, None]
  cps[0] = pltpu.async_copy(tgt.at[pl.ds(base, tpb)], b0, sem0)

  inf = jnp.full((_L,), jnp.inf, jnp.float32)
  ninf = jnp.full((_L,), -jnp.inf, jnp.float32)
  mins = [inf] * n_genes
  maxs = [ninf] * n_genes

  vpt = _TW // _L
  for b in range(nblk):
    cur = b % 2
    if b + 1 < nblk:
      nxt = (b + 1) % 2
      cps[nxt] = pltpu.async_copy(
          tgt.at[pl.ds(base + (b + 1) * tpb, tpb)], bufs[nxt], sems[nxt])
    cps[cur].wait()
    buf = bufs[cur]

    def body(i, carry):
      ti = i >> 3
      j = (i & 7) * _L
      out = []
      for g in range(n_genes):
        t = buf[ti, g, pl.ds(j, _L)]
        out.append(jnp.minimum(carry[g], t))
        out.append(jnp.maximum(carry[n_genes + g], t))
      return tuple(out[0::2]) + tuple(out[1::2])

    carry = lax.fori_loop(0, tpb * vpt, body, tuple(mins) + tuple(maxs))
    mins = list(carry[:n_genes])
    maxs = list(carry[n_genes:])

  tailcp.wait()
  pick = jnp.broadcast_to(wid < rem, (_L,))
  for g in range(n_genes):
    for j in range(vpt):
      t = tail[0, g, pl.ds(j * _L, _L)]
      mins[g] = jnp.minimum(mins[g], jnp.where(pick, t, inf))
      maxs[g] = jnp.maximum(maxs[g], jnp.where(pick, t, ninf))

  for g in range(n_genes):
    stmin[g, :] = mins[g]
    stmax[g, :] = maxs[g]
  pltpu.sync_copy(stmin, out.at[wid, 0])
  pltpu.sync_copy(stmax, out.at[wid, 1])


def _wmse_body(n_genes, kbins, tps, rem, tpb, nblk, num_tiles,
               pred, tgt, wts, minv, scalev, out,
               tb0, tb1, pb0, pb1, ttail, ptail, wv, mv, sv, stacc,
               ts0, ts1, ps0, ps1, tst, pst):
  wid = _wid()
  base = wid * tps + jnp.minimum(wid, rem)
  tbufs = (tb0, tb1)
  pbufs = (pb0, pb1)
  tsems = (ts0, ts1)
  psems = (ps0, ps1)

  pltpu.sync_copy(wts, wv)
  pltpu.sync_copy(minv, mv)
  pltpu.sync_copy(scalev, sv)
  m = [mv[g, :] for g in range(n_genes)]
  s = [sv[g, :] for g in range(n_genes)]
  gconst = [jnp.full((_L,), g, jnp.int32) for g in range(n_genes)]

  tailidx = jnp.minimum(base + tps, num_tiles - 1)
  ttcp = pltpu.async_copy(tgt.at[pl.ds(tailidx, 1)], ttail, tst)
  ptcp = pltpu.async_copy(pred.at[pl.ds(tailidx, 1)], ptail, pst)
  tcps = [None, None]
  pcps = [None, None]
  tcps[0] = pltpu.async_copy(tgt.at[pl.ds(base, tpb)], tb0, ts0)
  pcps[0] = pltpu.async_copy(pred.at[pl.ds(base, tpb)], pb0, ps0)

  zero = jnp.zeros((_L,), jnp.float32)
  accs = [zero] * n_genes
  kmax = kbins - 1
  vpt = _TW // _L

  for b in range(nblk):
    cur = b % 2
    if b + 1 < nblk:
      nxt = (b + 1) % 2
      off = base + (b + 1) * tpb
      tcps[nxt] = pltpu.async_copy(
          tgt.at[pl.ds(off, tpb)], tbufs[nxt], tsems[nxt])
      pcps[nxt] = pltpu.async_copy(
          pred.at[pl.ds(off, tpb)], pbufs[nxt], psems[nxt])
    tcps[cur].wait()
    pcps[cur].wait()
    tbuf = tbufs[cur]
    pbuf = pbufs[cur]

    def body(i, accs):
      ti = i >> 3
      j = (i & 7) * _L
      new = []
      for g in range(n_genes):
        t = tbuf[ti, g, pl.ds(j, _L)]
        p = pbuf[ti, g, pl.ds(j, _L)]
        d = p - t
        u = (t - m[g]) * s[g]
        bi = jnp.minimum(jnp.maximum(u.astype(jnp.int32), 0), kmax)
        w = plsc.load_gather(wv, [gconst[g], bi])
        new.append(accs[g] + d * d * w)
      return tuple(new)

    accs = list(lax.fori_loop(0, tpb * vpt, body, tuple(accs)))

  ttcp.wait()
  ptcp.wait()
  pick = jnp.broadcast_to(wid < rem, (_L,))
  for g in range(n_genes):
    for j in range(vpt):
      t = ttail[0, g, pl.ds(j * _L, _L)]
      p = ptail[0, g, pl.ds(j * _L, _L)]
      d = p - t
      u = (t - m[g]) * s[g]
      bi = jnp.minimum(jnp.maximum(u.astype(jnp.int32), 0), kmax)
      w = plsc.load_gather(wv, [gconst[g], bi])
      accs[g] = accs[g] + jnp.where(pick, d * d * w, zero)

  for g in range(n_genes):
    stacc[g, :] = accs[g]
  pltpu.sync_copy(stacc, out.at[wid])


@jax.jit
def kernel(pred, target, weights):
  n, n_genes = target.shape
  kbins = weights.shape[1]
  num_tiles = n // _TW
  tps, rem, tpb, nblk = _tile_split(num_tiles)

  mesh = plsc.VectorSubcoreMesh(core_axis_name="c", subcore_axis_name="s")
  t3 = target.T.reshape(n_genes, num_tiles, _TW).transpose(1, 0, 2)
  p3 = pred.T.reshape(n_genes, num_tiles, _TW).transpose(1, 0, 2)

  minmax = pl.kernel(
      functools.partial(_minmax_body, n_genes, tps, rem, tpb, nblk, num_tiles),
      out_type=jax.ShapeDtypeStruct((_NW, 2, n_genes, _L), jnp.float32),
      mesh=mesh,
      scratch_types=[
          pltpu.VMEM((tpb, n_genes, _TW), jnp.float32),
          pltpu.VMEM((tpb, n_genes, _TW), jnp.float32),
          pltpu.VMEM((1, n_genes, _TW), jnp.float32),
          pltpu.VMEM((n_genes, _L), jnp.float32),
          pltpu.VMEM((n_genes, _L), jnp.float32),
          pltpu.SemaphoreType.DMA,
          pltpu.SemaphoreType.DMA,
          pltpu.SemaphoreType.DMA,
      ],
      compiler_params=pltpu.CompilerParams(needs_layout_passes=False),
  )(t3)

  gmin = minmax[:, 0].min(axis=(0, 2))
  gmax = minmax[:, 1].max(axis=(0, 2))
  scale = kbins / (gmax - gmin)
  minv = jnp.broadcast_to(gmin[:, None], (n_genes, _L))
  scalev = jnp.broadcast_to(scale[:, None], (n_genes, _L))

  partial = pl.kernel(
      functools.partial(
          _wmse_body, n_genes, kbins, tps, rem, tpb, nblk, num_tiles),
      out_type=jax.ShapeDtypeStruct((_NW, n_genes, _L), jnp.float32),
      mesh=mesh,
      scratch_types=[
          pltpu.VMEM((tpb, n_genes, _TW), jnp.float32),
          pltpu.VMEM((tpb, n_genes, _TW), jnp.float32),
          pltpu.VMEM((tpb, n_genes, _TW), jnp.float32),
          pltpu.VMEM((tpb, n_genes, _TW), jnp.float32),
          pltpu.VMEM((1, n_genes, _TW), jnp.float32),
          pltpu.VMEM((1, n_genes, _TW), jnp.float32),
          pltpu.VMEM((n_genes, kbins), jnp.float32),
          pltpu.VMEM((n_genes, _L), jnp.float32),
          pltpu.VMEM((n_genes, _L), jnp.float32),
          pltpu.VMEM((n_genes, _L), jnp.float32),
          pltpu.SemaphoreType.DMA,
          pltpu.SemaphoreType.DMA,
          pltpu.SemaphoreType.DMA,
          pltpu.SemaphoreType.DMA,
          pltpu.SemaphoreType.DMA,
          pltpu.SemaphoreType.DMA,
      ],
      compiler_params=pltpu.CompilerParams(needs_layout_passes=False),
  )(p3, t3, weights, minv, scalev)

  return jnp.sum(partial) / (n_genes * n)


# speedup vs baseline: 53.3765x; 1.0650x over previous
import functools

import jax
import jax.numpy as jnp
from jax import lax
from jax.experimental import pallas as pl
from jax.experimental.pallas import tpu as pltpu
from jax.experimental.pallas import tpu_sc as plsc

_L = 16
_TW = 128
_NW = 32
_VPT = _TW // _L


def _wid():
  return lax.axis_index("s") * 2 + lax.axis_index("c")


def _block_sizes(tps, tpb):
  sizes = [tpb] * (tps // tpb)
  if tps % tpb:
    sizes.append(tps % tpb)
  return sizes


def _stream_blocks(srcs, bufs2, sems2, base, sizes, process):
  nsrc = len(srcs)
  cps = [[None, None] for _ in range(nsrc)]
  off = 0
  for k in range(nsrc):
    dst = bufs2[k][0]
    if sizes[0] != dst.shape[0]:
      dst = dst.at[pl.ds(0, sizes[0])]
    cps[k][0] = pltpu.async_copy(
        srcs[k].at[pl.ds(base, sizes[0])], dst, sems2[k][0])

  def run(carry):
    off = 0
    for b, sz in enumerate(sizes):
      cur = b % 2
      if b + 1 < len(sizes):
        nxt = (b + 1) % 2
        nsz = sizes[b + 1]
        noff = off + sz
        for k in range(nsrc):
          dst = bufs2[k][nxt]
          if nsz != dst.shape[0]:
            dst = dst.at[pl.ds(0, nsz)]
          cps[k][nxt] = pltpu.async_copy(
              srcs[k].at[pl.ds(base + noff, nsz)], dst, sems2[k][nxt])
      for k in range(nsrc):
        cps[k][cur].wait()
      carry = process([bufs2[k][cur] for k in range(nsrc)], sz, carry)
      off += sz
    return carry

  return run


def _minmax_body(n_genes, tps, rem, tpb, num_tiles,
                 tgt, out, b0, b1, tail, stmin, stmax, sem0, sem1, semt):
  wid = _wid()
  base = wid * tps + jnp.minimum(wid, rem)

  tailcp = pltpu.async_copy(
      tgt.at[pl.ds(jnp.minimum(base + tps, num_tiles - 1), 1)], tail, semt)

  inf = jnp.full((_L,), jnp.inf, jnp.float32)
  ninf = jnp.full((_L,), -jnp.inf, jnp.float32)

  def process(bufs, sz, carry):
    buf = bufs[0]

    def body(i, carry):
      ti = i >> 3
      j = (i & 7) * _L
      new = []
      for g in range(n_genes):
        t = buf[ti, g, pl.ds(j, _L)]
        new.append(jnp.minimum(carry[g], t))
        new.append(jnp.maximum(carry[n_genes + g], t))
      return tuple(new[0::2]) + tuple(new[1::2])

    return lax.fori_loop(0, sz * _VPT, body, carry, unroll=2)

  run = _stream_blocks([tgt], [(b0, b1)], [(sem0, sem1)], base,
                       _block_sizes(tps, tpb), process)
  carry = run((inf,) * n_genes + (ninf,) * n_genes)
  mins = list(carry[:n_genes])
  maxs = list(carry[n_genes:])

  tailcp.wait()
  pick = jnp.broadcast_to(wid < rem, (_L,))
  for g in range(n_genes):
    for j in range(_VPT):
      t = tail[0, g, pl.ds(j * _L, _L)]
      mins[g] = jnp.minimum(mins[g], jnp.where(pick, t, inf))
      maxs[g] = jnp.maximum(maxs[g], jnp.where(pick, t, ninf))

  for g in range(n_genes):
    stmin[g, :] = mins[g]
    stmax[g, :] = maxs[g]
  pltpu.sync_copy(stmin, out.at[wid, 0])
  pltpu.sync_copy(stmax, out.at[wid, 1])


def _wmse_body(n_genes, kbins, tps, rem, tpb, num_tiles,
               pred, tgt, wts, minmax, out,
               tb0, tb1, pb0, pb1, ttail, ptail, wv, mmv, stacc,
               ts0, ts1, ps0, ps1, tst, pst):
  wid = _wid()
  base = wid * tps + jnp.minimum(wid, rem)

  tailidx = jnp.minimum(base + tps, num_tiles - 1)
  ttcp = pltpu.async_copy(tgt.at[pl.ds(tailidx, 1)], ttail, tst)
  ptcp = pltpu.async_copy(pred.at[pl.ds(tailidx, 1)], ptail, pst)

  pltpu.sync_copy(wts, wv)
  pltpu.sync_copy(minmax, mmv)

  inf = jnp.full((_L,), jnp.inf, jnp.float32)
  ninf = jnp.full((_L,), -jnp.inf, jnp.float32)

  def fold(w, carry):
    new = []
    for g in range(n_genes):
      new.append(jnp.minimum(carry[g], mmv[w, 0, g, pl.ds(0, _L)]))
      new.append(jnp.maximum(carry[n_genes + g], mmv[w, 1, g, pl.ds(0, _L)]))
    return tuple(new[0::2]) + tuple(new[1::2])

  folded = lax.fori_loop(0, _NW, fold, (inf,) * n_genes + (ninf,) * n_genes)
  m = []
  s = []
  kvec = jnp.full((_L,), float(kbins), jnp.float32)
  for g in range(n_genes):
    mnv = jnp.full((_L,), jnp.min(folded[g]), jnp.float32)
    mxv = jnp.full((_L,), jnp.max(folded[n_genes + g]), jnp.float32)
    m.append(mnv)
    s.append(kvec / (mxv - mnv))
  gconst = [jnp.full((_L,), g, jnp.int32) for g in range(n_genes)]

  zero = jnp.zeros((_L,), jnp.float32)
  kmax = kbins - 1

  def process(bufs, sz, accs):
    tbuf, pbuf = bufs

    def body(i, accs):
      ti = i >> 3
      j = (i & 7) * _L
      new = []
      for g in range(n_genes):
        t = tbuf[ti, g, pl.ds(j, _L)]
        p = pbuf[ti, g, pl.ds(j, _L)]
        d = p - t
        u = (t - m[g]) * s[g]
        bi = jnp.minimum(jnp.maximum(u.astype(jnp.int32), 0), kmax)
        w = plsc.load_gather(wv, [gconst[g], bi])
        new.append(accs[g] + d * d * w)
      return tuple(new)

    return lax.fori_loop(0, sz * _VPT, body, accs, unroll=2)

  run = _stream_blocks([tgt, pred], [(tb0, tb1), (pb0, pb1)],
                       [(ts0, ts1), (ps0, ps1)], base,
                       _block_sizes(tps, tpb), process)
  accs = list(run((zero,) * n_genes))

  ttcp.wait()
  ptcp.wait()
  pick = jnp.broadcast_to(wid < rem, (_L,))
  for g in range(n_genes):
    for j in range(_VPT):
      t = ttail[0, g, pl.ds(j * _L, _L)]
      p = ptail[0, g, pl.ds(j * _L, _L)]
      d = p - t
      u = (t - m[g]) * s[g]
      bi = jnp.minimum(jnp.maximum(u.astype(jnp.int32), 0), kmax)
      w = plsc.load_gather(wv, [gconst[g], bi])
      accs[g] = accs[g] + jnp.where(pick, d * d * w, zero)

  for g in range(n_genes):
    stacc[g, :] = accs[g]
  pltpu.sync_copy(stacc, out.at[wid])


@jax.jit
def kernel(pred, target, weights):
  n, n_genes = target.shape
  kbins = weights.shape[1]
  num_tiles = n // _TW
  tps, rem = divmod(num_tiles, _NW)
  tpb1 = 61 if tps % 61 == 0 else 32
  tpb2 = 32

  mesh = plsc.VectorSubcoreMesh(core_axis_name="c", subcore_axis_name="s")
  t3 = target.T.reshape(n_genes, num_tiles, _TW).transpose(1, 0, 2)
  p3 = pred.T.reshape(n_genes, num_tiles, _TW).transpose(1, 0, 2)

  minmax = pl.kernel(
      functools.partial(_minmax_body, n_genes, tps, rem, tpb1, num_tiles),
      out_type=jax.ShapeDtypeStruct((_NW, 2, n_genes, _L), jnp.float32),
      mesh=mesh,
      scratch_types=[
          pltpu.VMEM((tpb1, n_genes, _TW), jnp.float32),
          pltpu.VMEM((tpb1, n_genes, _TW), jnp.float32),
          pltpu.VMEM((1, n_genes, _TW), jnp.float32),
          pltpu.VMEM((n_genes, _L), jnp.float32),
          pltpu.VMEM((n_genes, _L), jnp.float32),
          pltpu.SemaphoreType.DMA,
          pltpu.SemaphoreType.DMA,
          pltpu.SemaphoreType.DMA,
      ],
      compiler_params=pltpu.CompilerParams(needs_layout_passes=False),
  )(t3)

  partial = pl.kernel(
      functools.partial(
          _wmse_body, n_genes, kbins, tps, rem, tpb2, num_tiles),
      out_type=jax.ShapeDtypeStruct((_NW, n_genes, _L), jnp.float32),
      mesh=mesh,
      scratch_types=[
          pltpu.VMEM((tpb2, n_genes, _TW), jnp.float32),
          pltpu.VMEM((tpb2, n_genes, _TW), jnp.float32),
          pltpu.VMEM((tpb2, n_genes, _TW), jnp.float32),
          pltpu.VMEM((tpb2, n_genes, _TW), jnp.float32),
          pltpu.VMEM((1, n_genes, _TW), jnp.float32),
          pltpu.VMEM((1, n_genes, _TW), jnp.float32),
          pltpu.VMEM((n_genes, kbins), jnp.float32),
          pltpu.VMEM((_NW, 2, n_genes, _L), jnp.float32),
          pltpu.VMEM((n_genes, _L), jnp.float32),
          pltpu.SemaphoreType.DMA,
          pltpu.SemaphoreType.DMA,
          pltpu.SemaphoreType.DMA,
          pltpu.SemaphoreType.DMA,
          pltpu.SemaphoreType.DMA,
          pltpu.SemaphoreType.DMA,
      ],
      compiler_params=pltpu.CompilerParams(needs_layout_passes=False),
  )(p3, t3, weights, minmax)

  return jnp.sum(partial) / (n_genes * n)
